# Initial kernel scaffold; baseline (speedup 1.0000x reference)
import jax
import jax.numpy as jnp
from jax.experimental import pallas as pl


def kernel(x, edge_attr, W_ee, b_ee, Wrel1, Wroot1, b1, Wrel2, Wroot2, b2, Wrel3, Wroot3, b3, Wl1, Wr1, We1, att1, bg1, Wl2, Wr2, We2, att2, bg2, Wfc, bfc, gamma, beta, edge_index, edge_type, batch):
    raise NotImplementedError("write your pallas kernel here")


# speedup vs baseline: 6.5479x; 6.5479x over previous
import functools

import jax
import jax.numpy as jnp
from jax import lax
from jax.experimental import pallas as pl
from jax.experimental.pallas import tpu as pltpu
from jax.experimental.pallas import tpu_sc as plsc

N = 10000
E = 160000
D = 78
NREL = 5
G = 64
N5 = N * NREL

NC = 2
NS = 16
CE = 40
NJ32 = E // (NC * NS) // CE
NJ16 = E // NS // CE
RS5P = 3128
N5P = NS * RS5P
RSNP = 632
NP = NS * RSNP

f32 = jnp.float32
i32 = jnp.int32


def _mesh():
  return plsc.VectorSubcoreMesh(core_axis_name="c", subcore_axis_name="s")


_SC_PARAMS = pltpu.CompilerParams(use_tc_tiling_on_sc=False)


def _make_count():
  @functools.partial(
      pl.kernel,
      out_type=jax.ShapeDtypeStruct((N5P, 16), f32),
      mesh=_mesh(),
      compiler_params=_SC_PARAMS,
      scratch_types=[
          pltpu.VMEM((NJ16, CE), i32),
          pltpu.VMEM((CE, 16), f32),
          pltpu.VMEM_SHARED((N5P, 16), f32),
      ],
  )
  def k(dst5_h, ones_h, z5_h, cnt_h, idxb, onesb, tbl):
    cid = lax.axis_index("c")
    sid = lax.axis_index("s")

    @pl.when(cid == 0)
    def _():
      pltpu.sync_copy(ones_h, onesb)
      pltpu.sync_copy(dst5_h.at[sid], idxb)
      base = sid * RS5P
      pltpu.sync_copy(z5_h, tbl.at[pl.ds(base, RS5P)])
      plsc.subcore_barrier()

      def jbody(j, _):
        pltpu.sync_copy(onesb, tbl.at[idxb.at[j]], add=True)
        return 0

      lax.fori_loop(0, NJ16, jbody, 0)
      plsc.subcore_barrier()
      pltpu.sync_copy(tbl.at[pl.ds(base, RS5P)], cnt_h.at[pl.ds(base, RS5P)])

  return k


def _make_agg(nk):
  @functools.partial(
      pl.kernel,
      out_type=[jax.ShapeDtypeStruct((N5P, 16), f32) for _ in range(nk)],
      mesh=_mesh(),
      compiler_params=_SC_PARAMS,
      scratch_types=[
          pltpu.VMEM((NJ16, CE), i32),
          pltpu.VMEM((NJ16, CE), i32),
          pltpu.VMEM((CE, 16), f32),
          pltpu.VMEM_SHARED((N5P, 16), f32),
          pltpu.SemaphoreType.DMA,
      ],
  )
  def k(*refs):
    hs = refs[:nk]
    src_h, dst5_h, z5_h = refs[nk:nk + 3]
    outs = refs[nk + 3:nk + 3 + nk]
    idxs, idxd, gbuf, tbl, sem = refs[nk + 3 + nk:]
    cid = lax.axis_index("c")
    sid = lax.axis_index("s")
    pltpu.sync_copy(src_h.at[sid], idxs)
    pltpu.sync_copy(dst5_h.at[sid], idxd)

    def do_chunk(kk):
      base = sid * RS5P
      pltpu.sync_copy(z5_h, tbl.at[pl.ds(base, RS5P)])
      plsc.subcore_barrier()

      def jbody(j, _):
        pltpu.async_copy(hs[kk].at[idxs.at[j]], gbuf, sem).wait()
        pltpu.sync_copy(gbuf, tbl.at[idxd.at[j]], add=True)
        return 0

      lax.fori_loop(0, NJ16, jbody, 0)
      plsc.subcore_barrier()
      pltpu.sync_copy(tbl.at[pl.ds(base, RS5P)],
                      outs[kk].at[pl.ds(base, RS5P)])
      plsc.subcore_barrier()

    @pl.when(cid == 0)
    def _():
      for kk in range(0, nk, 2):
        do_chunk(kk)

    @pl.when(cid == 1)
    def _():
      for kk in range(1, nk, 2):
        do_chunk(kk)

  return k


def _make_score(H, Cp):
  nkc = Cp // 16

  @functools.partial(
      pl.kernel,
      out_type=[jax.ShapeDtypeStruct((E, 16), f32) for _ in range(H)],
      mesh=_mesh(),
      compiler_params=_SC_PARAMS,
      scratch_types=[
          pltpu.VMEM((8, Cp), f32),
          pltpu.VMEM((NJ32, CE), i32),
          pltpu.VMEM((NJ32, CE), i32),
          pltpu.VMEM((CE, Cp), f32),
          pltpu.VMEM((CE, Cp), f32),
          pltpu.VMEM((CE, Cp), f32),
          pltpu.VMEM((CE, 16), f32),
          pltpu.SemaphoreType.DMA,
          pltpu.SemaphoreType.DMA,
      ],
  )
  def k(*refs):
    xls = refs[:H]
    xrs = refs[H:2 * H]
    ems = refs[2 * H:3 * H]
    att_h, src_h, dst_h = refs[3 * H:3 * H + 3]
    sparts = refs[3 * H + 3:3 * H + 3 + H]
    attb, idxs, idxd, xlb, xrb, emb, sbuf, sem1, sem2 = refs[3 * H + 3 + H:]
    cid = lax.axis_index("c")
    sid = lax.axis_index("s")
    wid = sid * NC + cid

    pltpu.sync_copy(att_h, attb)
    pltpu.sync_copy(src_h.at[wid], idxs)
    pltpu.sync_copy(dst_h.at[wid], idxd)
    for h in range(H):

      def jbody(j, _):
        row0 = wid * (E // (NC * NS)) + j * CE
        d1 = pltpu.async_copy(xls[h].at[idxs.at[j]], xlb, sem1)
        d2 = pltpu.async_copy(xrs[h].at[idxd.at[j]], xrb, sem2)
        pltpu.sync_copy(ems[h].at[pl.ds(row0, CE)], emb)
        d1.wait()
        d2.wait()

        def rbody(r, _):
          acc = jnp.zeros((16,), f32)
          for kk in range(nkc):
            z = (xlb[r, pl.ds(kk * 16, 16)] + xrb[r, pl.ds(kk * 16, 16)]
                 + emb[r, pl.ds(kk * 16, 16)])
            z = jnp.where(z > 0, z, 0.2 * z)
            acc = acc + z * attb[h, pl.ds(kk * 16, 16)]
          sbuf[r] = acc
          return 0

        lax.fori_loop(0, CE, rbody, 0)
        pltpu.sync_copy(sbuf, sparts[h].at[pl.ds(row0, CE)])
        return 0

      lax.fori_loop(0, NJ32, jbody, 0)

  return k


def _make_segsum(H):
  @functools.partial(
      pl.kernel,
      out_type=[jax.ShapeDtypeStruct((NP, 16), f32) for _ in range(H)],
      mesh=_mesh(),
      compiler_params=_SC_PARAMS,
      scratch_types=[
          pltpu.VMEM((NJ16, CE), i32),
          pltpu.VMEM((CE, 16), f32),
          pltpu.VMEM_SHARED((NP, 16), f32),
      ],
  )
  def k(*refs):
    e_hs = refs[:H]
    dst_h, z5_h = refs[H:H + 2]
    dens = refs[H + 2:H + 2 + H]
    idxd, ebuf, tbl = refs[H + 2 + H:]
    cid = lax.axis_index("c")
    sid = lax.axis_index("s")
    pltpu.sync_copy(dst_h.at[sid], idxd)

    def job(h):
      base = sid * RSNP
      pltpu.sync_copy(z5_h.at[pl.ds(0, RSNP)], tbl.at[pl.ds(base, RSNP)])
      plsc.subcore_barrier()

      def jbody(j, _):
        row0 = sid * (E // NS) + j * CE
        pltpu.sync_copy(e_hs[h].at[pl.ds(row0, CE)], ebuf)
        pltpu.sync_copy(ebuf, tbl.at[idxd.at[j]], add=True)
        return 0

      lax.fori_loop(0, NJ16, jbody, 0)
      plsc.subcore_barrier()
      pltpu.sync_copy(tbl.at[pl.ds(base, RSNP)],
                      dens[h].at[pl.ds(base, RSNP)])
      plsc.subcore_barrier()

    @pl.when(cid == 0)
    def _():
      for h in range(H // 2):
        job(h)

    @pl.when(cid == 1)
    def _():
      for h in range(H // 2, H):
        job(h)

  return k


def _make_wsum(H, Cp):
  nkc = Cp // 16

  @functools.partial(
      pl.kernel,
      out_type=[jax.ShapeDtypeStruct((NP, Cp), f32) for _ in range(H)],
      mesh=_mesh(),
      compiler_params=_SC_PARAMS,
      scratch_types=[
          pltpu.VMEM((NJ16, CE), i32),
          pltpu.VMEM((NJ16, CE), i32),
          pltpu.VMEM((CE, 16), f32),
          pltpu.VMEM((CE, 16), f32),
          pltpu.VMEM((CE, Cp), f32),
          pltpu.VMEM_SHARED((NP, Cp), f32),
          pltpu.SemaphoreType.DMA,
          pltpu.SemaphoreType.DMA,
      ],
  )
  def k(*refs):
    xls = refs[:H]
    dens = refs[H:2 * H]
    e_hs = refs[2 * H:3 * H]
    src_h, dst_h, zc_h = refs[3 * H:3 * H + 3]
    outs = refs[3 * H + 3:3 * H + 3 + H]
    idxs, idxd, dbuf, ebuf, xbuf, tbl, sem1, sem2 = refs[3 * H + 3 + H:]
    cid = lax.axis_index("c")
    sid = lax.axis_index("s")
    pltpu.sync_copy(src_h.at[sid], idxs)
    pltpu.sync_copy(dst_h.at[sid], idxd)

    def job(h):
      base = sid * RSNP
      pltpu.sync_copy(zc_h, tbl.at[pl.ds(base, RSNP)])
      plsc.subcore_barrier()

      def jbody(j, _):
        row0 = sid * (E // NS) + j * CE
        d1 = pltpu.async_copy(dens[h].at[idxd.at[j]], dbuf, sem1)
        d2 = pltpu.async_copy(xls[h].at[idxs.at[j]], xbuf, sem2)
        pltpu.sync_copy(e_hs[h].at[pl.ds(row0, CE)], ebuf)
        d1.wait()
        d2.wait()

        def rbody(r, _):
          av = ebuf[r] / jnp.maximum(dbuf[r], 1e-30)
          for kk in range(nkc):
            xbuf[r, pl.ds(kk * 16, 16)] = xbuf[r, pl.ds(kk * 16, 16)] * av
          return 0

        lax.fori_loop(0, CE, rbody, 0)
        pltpu.sync_copy(xbuf, tbl.at[idxd.at[j]], add=True)
        return 0

      lax.fori_loop(0, NJ16, jbody, 0)
      plsc.subcore_barrier()
      pltpu.sync_copy(tbl.at[pl.ds(base, RSNP)],
                      outs[h].at[pl.ds(base, RSNP)])
      plsc.subcore_barrier()

    @pl.when(cid == 0)
    def _():
      for h in range(H // 2):
        job(h)

    @pl.when(cid == 1)
    def _():
      for h in range(H // 2, H):
        job(h)

  return k


def _dot(a, b):
  return lax.dot_general(a, b, (((1,), (0,)), ((), ())),
                         preferred_element_type=f32)


def _mm(A, W, bias=None, relu=False, bm=400):
  M, K = A.shape
  Nc = W.shape[1]
  grid = (M // bm,)
  in_specs = [
      pl.BlockSpec((bm, K), lambda i: (i, 0)),
      pl.BlockSpec((K, Nc), lambda i: (0, 0)),
  ]
  if bias is not None:
    in_specs.append(pl.BlockSpec((8, Nc), lambda i: (0, 0)))

  def body(*refs):
    a_ref, w_ref = refs[0], refs[1]
    o_ref = refs[-1]
    acc = _dot(a_ref[...], w_ref[...])
    if bias is not None:
      acc = acc + refs[2][0:1, :]
    if relu:
      acc = jnp.maximum(acc, 0.0)
    o_ref[...] = acc

  fn = pl.pallas_call(
      body, grid=grid, in_specs=in_specs,
      out_specs=pl.BlockSpec((bm, Nc), lambda i: (i, 0)),
      out_shape=jax.ShapeDtypeStruct((M, Nc), f32))
  args = (A, W) if bias is None else (A, W, bias)
  return fn(*args)


def _mm_multi(A, Ws, bm=400):
  M, K = A.shape
  Cp = Ws[0].shape[1]
  nw = len(Ws)

  def body(*refs):
    a = refs[0]
    ws = refs[1:1 + nw]
    outs = refs[1 + nw:]
    av = a[...]
    for h in range(nw):
      outs[h][...] = _dot(av, ws[h][...])

  return pl.pallas_call(
      body, grid=(M // bm,),
      in_specs=[pl.BlockSpec((bm, K), lambda i: (i, 0))] +
               [pl.BlockSpec((K, Cp), lambda i: (0, 0))] * nw,
      out_specs=[pl.BlockSpec((bm, Cp), lambda i: (i, 0))] * nw,
      out_shape=[jax.ShapeDtypeStruct((M, Cp), f32)] * nw)(A, *Ws)


def _rgcn_mm(h_p, Wroot_p, aggf, cntf, Wrel_f, brep, bm=400):
  M, K1 = h_p.shape
  K2 = aggf.shape[1]
  Nc = Wroot_p.shape[1]

  def body(a1, w1, a2, c2, w2, b, o):
    mean = a2[...] / jnp.maximum(c2[...], 1.0)
    acc = _dot(a1[...], w1[...]) + _dot(mean, w2[...]) + b[0:1, :]
    o[...] = jnp.maximum(acc, 0.0)

  return pl.pallas_call(
      body, grid=(M // bm,),
      in_specs=[
          pl.BlockSpec((bm, K1), lambda i: (i, 0)),
          pl.BlockSpec((K1, Nc), lambda i: (0, 0)),
          pl.BlockSpec((bm, K2), lambda i: (i, 0)),
          pl.BlockSpec((bm, K2), lambda i: (i, 0)),
          pl.BlockSpec((K2, Nc), lambda i: (0, 0)),
          pl.BlockSpec((8, Nc), lambda i: (0, 0)),
      ],
      out_specs=pl.BlockSpec((bm, Nc), lambda i: (i, 0)),
      out_shape=jax.ShapeDtypeStruct((M, Nc), f32))(
          h_p, Wroot_p, aggf, cntf, Wrel_f, brep)


def _bias_relu(A, brep, bm=400):
  M, W = A.shape

  def body(a, b, o):
    o[...] = jnp.maximum(a[...] + b[0:1, :], 0.0)

  return pl.pallas_call(
      body, grid=(M // bm,),
      in_specs=[pl.BlockSpec((bm, W), lambda i: (i, 0)),
                pl.BlockSpec((8, W), lambda i: (0, 0))],
      out_specs=pl.BlockSpec((bm, W), lambda i: (i, 0)),
      out_shape=jax.ShapeDtypeStruct((M, W), f32))(A, brep)


def _colmax(sparts, ones16, bm=1600):
  H = len(sparts)

  def body(*refs):
    i = pl.program_id(0)
    ss = refs[:H]
    b = refs[H]
    o = refs[H + 1]

    @pl.when(i == 0)
    def _():
      o[...] = jnp.full((8, H * 16), -3e38, f32)

    for h in range(H):
      srep = _dot(ss[h][...], b[...])
      m = jnp.max(srep, axis=0, keepdims=True)
      o[:, h * 16:(h + 1) * 16] = jnp.maximum(
          o[:, h * 16:(h + 1) * 16], jnp.broadcast_to(m, (8, 16)))

  return pl.pallas_call(
      body, grid=(E // bm,),
      in_specs=[pl.BlockSpec((bm, 16), lambda i: (i, 0))] * H +
               [pl.BlockSpec((16, 16), lambda i: (0, 0))],
      out_specs=pl.BlockSpec((8, H * 16), lambda i: (0, 0)),
      out_shape=jax.ShapeDtypeStruct((8, H * 16), f32))(*sparts, ones16)


def _exp_e(sparts, ones16, mrep, bm=1600):
  H = len(sparts)

  def body(*refs):
    ss = refs[:H]
    b = refs[H]
    m = refs[H + 1]
    outs = refs[H + 2:]
    for h in range(H):
      srep = _dot(ss[h][...], b[...])
      outs[h][...] = jnp.exp(srep - m[0:1, h * 16:(h + 1) * 16])

  return pl.pallas_call(
      body, grid=(E // bm,),
      in_specs=[pl.BlockSpec((bm, 16), lambda i: (i, 0))] * H +
               [pl.BlockSpec((16, 16), lambda i: (0, 0)),
                pl.BlockSpec((8, H * 16), lambda i: (0, 0))],
      out_specs=[pl.BlockSpec((bm, 16), lambda i: (i, 0))] * H,
      out_shape=[jax.ShapeDtypeStruct((E, 16), f32)] * H)(
          *sparts, ones16, mrep)


WIN = 512


def _pool(x, batch2d, starts):
  W = x.shape[1]

  def body(st_ref, x_ref, b_ref, o_ref):
    def gbody(g, _):
      s0 = pl.multiple_of((jnp.minimum(st_ref[g], N - WIN) // 8) * 8, 8)
      xw = x_ref[pl.ds(s0, WIN), :]
      bw = b_ref[pl.ds(s0, WIN), 0:1]
      mask = bw == g
      mx = jnp.max(jnp.where(mask, xw, f32(-3e38)), axis=0, keepdims=True)
      mx = jnp.where(mx <= f32(-1e38), 0.0, mx)
      sm = jnp.sum(jnp.where(mask, xw, 0.0), axis=0, keepdims=True)
      cnt = (st_ref[g + 1] - st_ref[g]).astype(f32)
      mean = sm / jnp.maximum(cnt, 1.0)
      o_ref[pl.ds(g, 1), :] = jnp.concatenate([mx, mean], axis=1)
      return 0

    lax.fori_loop(0, G, gbody, 0)

  return pl.pallas_call(
      body,
      in_specs=[pl.BlockSpec(memory_space=pltpu.SMEM),
                pl.BlockSpec((N, W), lambda: (0, 0)),
                pl.BlockSpec((N, 128), lambda: (0, 0))],
      out_specs=pl.BlockSpec((G, 2 * W), lambda: (0, 0)),
      out_shape=jax.ShapeDtypeStruct((G, 2 * W), f32))(starts, x, batch2d)


def _fc_ln(h, Wfc, brep, grep, berep):
  def body(h_ref, w_ref, b_ref, g_ref, be_ref, o_ref):
    z = _dot(h_ref[...], w_ref[...]) + b_ref[0:1, :]
    z = jnp.maximum(z, 0.0)
    mu = jnp.mean(z, axis=-1, keepdims=True)
    d = z - mu
    var = jnp.mean(d * d, axis=-1, keepdims=True)
    o_ref[...] = d / jnp.sqrt(var + 1e-5) * g_ref[0:1, :] + be_ref[0:1, :]

  return pl.pallas_call(
      body, out_shape=jax.ShapeDtypeStruct((G, 1024), f32))(
          h, Wfc, brep, grep, berep)


def _pad2(a, r, c):
  return jnp.pad(a, ((0, r - a.shape[0]), (0, c - a.shape[1])))


def _rep8(v, w):
  return jnp.broadcast_to(jnp.pad(v, (0, w - v.shape[0]))[None, :], (8, w))


def _headpad_cols(Wm, H, C, Cp, rpad):
  K = Wm.shape[0]
  w = jnp.pad(Wm.reshape(K, H, C), ((0, rpad - K), (0, 0), (0, Cp - C)))
  return w.reshape(rpad, H * Cp)


def _headpad_rows(Wm, Hi, Ci, Cip):
  Nc = Wm.shape[1]
  w = jnp.pad(Wm.reshape(Hi, Ci, Nc), ((0, 0), (0, Cip - Ci), (0, 0)))
  return w.reshape(Hi * Cip, Nc)


def kernel(x, edge_attr, W_ee, b_ee, Wrel1, Wroot1, b1, Wrel2, Wroot2, b2,
           Wrel3, Wroot3, b3, Wl1, Wr1, We1, att1, bg1, Wl2, Wr2, We2, att2,
           bg2, Wfc, bfc, gamma, beta, edge_index, edge_type, batch):
  src = edge_index[0].astype(i32)
  dst = edge_index[1].astype(i32)
  dst5 = dst * NREL + edge_type.astype(i32)

  src32 = src.reshape(NC * NS, NJ32, CE)
  dst32 = dst.reshape(NC * NS, NJ32, CE)
  src16 = src.reshape(NS, NJ16, CE)
  dst16 = dst.reshape(NS, NJ16, CE)
  dst5_16 = dst5.reshape(NS, NJ16, CE)

  z5 = jnp.zeros((RS5P, 16), f32)
  ones_in = jnp.ones((CE, 16), f32)
  ones16 = jnp.ones((16, 16), f32)

  cnt = _make_count()(dst5_16, ones_in, z5)
  cnt5 = cnt[:N5, 0].reshape(N, NREL)

  x_p = _pad2(x, N, 80)
  agg5 = _make_agg(5)
  agg10 = _make_agg(10)

  def rgcn(h_p, Pin, agg, Wroot, b, Wrel, Pout):
    hs = [h_p[:, 16 * kk:16 * kk + 16] for kk in range(Pin // 16)]
    sums = agg(*hs, src16, dst5_16, z5)
    aggf = jnp.concatenate(
        [s[:N5].reshape(N, NREL, 16) for s in sums], axis=2
    ).reshape(N, NREL * Pin)
    cntf = jnp.broadcast_to(cnt5[:, :, None], (N, NREL, Pin)).reshape(
        N, NREL * Pin)
    Din, Dout = Wroot.shape
    Wroot_p = _pad2(Wroot, Pin, Pout)
    Wrel_p = jnp.pad(Wrel, ((0, 0), (0, Pin - Din), (0, Pout - Dout)))
    Wrel_f = Wrel_p.reshape(NREL * Pin, Pout)
    return _rgcn_mm(h_p, Wroot_p, aggf, cntf, Wrel_f, _rep8(b, Pout))

  h1 = rgcn(x_p, 80, agg5, Wroot1, b1, Wrel1, 80)
  h2 = rgcn(h1, 80, agg5, Wroot2, b2, Wrel2, 160)
  x3 = rgcn(h2, 160, agg10, Wroot3, b3, Wrel3, 320)

  batch_i = batch.astype(i32)
  starts = jnp.searchsorted(batch_i, jnp.arange(G + 1, dtype=i32)).astype(i32)
  batch2d = jnp.broadcast_to(batch_i[:, None], (N, 128))
  p1 = _pool(x3, batch2d, starts)

  ee = _mm(edge_attr, _pad2(W_ee, 8, 128), bias=_rep8(b_ee, 128), relu=True,
           bm=1600)

  def gat(x_in, Pi, H, C, Cp, Wl, Wr, We, att, bg, Hi, Ci):
    Wl_r = _headpad_rows(Wl, Hi, Ci, Pi // Hi)
    Wr_r = _headpad_rows(Wr, Hi, Ci, Pi // Hi)
    Wl_hp = _headpad_cols(Wl_r, H, C, Cp, Pi)
    Wr_hp = _headpad_cols(Wr_r, H, C, Cp, Pi)
    We_hp = _headpad_cols(We, H, C, Cp, 128)
    Wlr_heads = ([Wl_hp[:, h * Cp:(h + 1) * Cp] for h in range(H)] +
                 [Wr_hp[:, h * Cp:(h + 1) * Cp] for h in range(H)])
    xlr = _mm_multi(x_in, Wlr_heads)
    xls, xrs = xlr[:H], xlr[H:]
    ems = _mm_multi(ee, [We_hp[:, h * Cp:(h + 1) * Cp] for h in range(H)],
                    bm=1600)
    att_p = jnp.pad(att, ((0, 8 - H), (0, Cp - C)))
    sparts = _make_score(H, Cp)(*xls, *xrs, *ems, att_p, src32, dst32)
    mrep = _colmax(sparts, ones16)
    e_hs = _exp_e(sparts, ones16, mrep)
    dens = _make_segsum(H)(*e_hs, dst16, z5)
    zc = jnp.zeros((RSNP, Cp), f32)
    outg = _make_wsum(H, Cp)(*xls, *dens, *e_hs, src16, dst16, zc)
    outg_cat = jnp.concatenate([o[:N] for o in outg], axis=1)
    bg_hp = jnp.pad(bg.reshape(H, C), ((0, 0), (0, Cp - C))).reshape(H * Cp)
    return _bias_relu(outg_cat, jnp.broadcast_to(bg_hp[None, :], (8, H * Cp)))

  x2 = gat(x_p, 80, 2, D, 80, Wl1, Wr1, We1, att1, bg1, 1, 78)
  x2b = gat(x2, 160, 4, 2 * D, 160, Wl2, Wr2, We2, att2, bg2, 2, 78)
  p2 = _pool(x2b, batch2d, starts)

  p1mx = p1[:, 0:312]
  p1me = p1[:, 320:632]
  p2mx = jnp.concatenate([p2[:, h * 160:h * 160 + 156] for h in range(4)], 1)
  p2me = jnp.concatenate(
      [p2[:, 640 + h * 160:640 + h * 160 + 156] for h in range(4)], 1)
  hcat = jnp.concatenate([p1mx, p1me, p2mx, p2me], axis=1)
  return _fc_ln(hcat, Wfc, _rep8(bfc, 1024), _rep8(gamma, 1024),
                _rep8(beta, 1024))


# speedup vs baseline: 7.7744x; 1.1873x over previous
import functools

import jax
import jax.numpy as jnp
from jax import lax
from jax.experimental import pallas as pl
from jax.experimental.pallas import tpu as pltpu
from jax.experimental.pallas import tpu_sc as plsc

N = 10000
E = 160000
D = 78
NREL = 5
G = 64
N5 = N * NREL

NC = 2
NS = 16
CE = 40
CE16 = 80
UN = 5
NJ32 = E // (NC * NS) // CE
NJ16 = E // NS // CE16
RS5P = 3128
N5P = NS * RS5P
RSNP = 632
NP = NS * RSNP

f32 = jnp.float32
i32 = jnp.int32


def _mesh():
  return plsc.VectorSubcoreMesh(core_axis_name="c", subcore_axis_name="s")


_SC_PARAMS = pltpu.CompilerParams(use_tc_tiling_on_sc=False)


def _make_count():
  @functools.partial(
      pl.kernel,
      out_type=[jax.ShapeDtypeStruct((N5P, 16), f32) for _ in range(NC)],
      mesh=_mesh(),
      compiler_params=_SC_PARAMS,
      scratch_types=[
          pltpu.VMEM((NJ16, CE16), i32),
          pltpu.VMEM((CE16, 16), f32),
          pltpu.VMEM_SHARED((N5P, 16), f32),
      ],
  )
  def k(dst5_h, ones_h, z5_h, cntA_h, cntB_h, idxb, onesb, tbl):
    cid = lax.axis_index("c")
    sid = lax.axis_index("s")
    pltpu.sync_copy(ones_h, onesb)
    pltpu.sync_copy(dst5_h.at[sid], idxb)
    base = sid * RS5P
    pltpu.sync_copy(z5_h, tbl.at[pl.ds(base, RS5P)])
    plsc.subcore_barrier()

    def jbody(j, _):
      pltpu.sync_copy(onesb, tbl.at[idxb.at[j]], add=True)
      return 0

    def scan_and_out(lo, hi, out_h):
      lax.fori_loop(lo, hi, jbody, 0)
      plsc.subcore_barrier()
      pltpu.sync_copy(tbl.at[pl.ds(base, RS5P)], out_h.at[pl.ds(base, RS5P)])

    @pl.when(cid == 0)
    def _():
      scan_and_out(0, NJ16 // 2, cntA_h)

    @pl.when(cid == 1)
    def _():
      scan_and_out(NJ16 // 2, NJ16, cntB_h)

  return k


def _make_agg(nk):
  @functools.partial(
      pl.kernel,
      out_type=[jax.ShapeDtypeStruct((N5P, 16), f32) for _ in range(nk)],
      mesh=_mesh(),
      compiler_params=_SC_PARAMS,
      scratch_types=[
          pltpu.VMEM((NJ16, CE16), i32),
          pltpu.VMEM((NJ16, CE16), i32),
      ] + [pltpu.VMEM((CE16, 16), f32) for _ in range(UN)] + [
          pltpu.VMEM_SHARED((N5P, 16), f32),
      ] + [pltpu.SemaphoreType.DMA] * (2 * UN),
  )
  def k(*refs):
    hs = refs[:nk]
    src_h, dst5_h, z5_h = refs[nk:nk + 3]
    outs = refs[nk + 3:nk + 3 + nk]
    rest = refs[nk + 3 + nk:]
    idxs, idxd = rest[0], rest[1]
    gbufs = rest[2:2 + UN]
    tbl = rest[2 + UN]
    gsems = rest[3 + UN:3 + 2 * UN]
    ssems = rest[3 + 2 * UN:3 + 3 * UN]
    cid = lax.axis_index("c")
    sid = lax.axis_index("s")
    pltpu.sync_copy(src_h.at[sid], idxs)
    pltpu.sync_copy(dst5_h.at[sid], idxd)

    def do_chunk(kk):
      base = sid * RS5P
      pltpu.sync_copy(z5_h, tbl.at[pl.ds(base, RS5P)])
      plsc.subcore_barrier()

      def jbody(jg, _):
        gs = [pltpu.async_copy(hs[kk].at[idxs.at[jg * UN + u]], gbufs[u],
                               gsems[u]) for u in range(UN)]
        ss = []
        for u in range(UN):
          gs[u].wait()
          ss.append(pltpu.async_copy(gbufs[u], tbl.at[idxd.at[jg * UN + u]],
                                     ssems[u], add=True))
        for s in ss:
          s.wait()
        return 0

      lax.fori_loop(0, NJ16 // UN, jbody, 0)
      plsc.subcore_barrier()
      pltpu.sync_copy(tbl.at[pl.ds(base, RS5P)],
                      outs[kk].at[pl.ds(base, RS5P)])
      plsc.subcore_barrier()

    @pl.when(cid == 0)
    def _():
      for kk in range(0, nk, 2):
        do_chunk(kk)

    @pl.when(cid == 1)
    def _():
      for kk in range(1, nk, 2):
        do_chunk(kk)

  return k


def _make_score(H, Cp):
  nkc = Cp // 16

  @functools.partial(
      pl.kernel,
      out_type=[jax.ShapeDtypeStruct((E, 16), f32) for _ in range(H)],
      mesh=_mesh(),
      compiler_params=_SC_PARAMS,
      scratch_types=[
          pltpu.VMEM((8, Cp), f32),
          pltpu.VMEM((NJ32, CE), i32),
          pltpu.VMEM((NJ32, CE), i32),
          pltpu.VMEM((UN * CE, Cp), f32),
      ] + [pltpu.VMEM((CE, Cp), f32) for _ in range(2 * UN)] +
      [pltpu.VMEM((CE, 16), f32) for _ in range(UN)] +
      [pltpu.SemaphoreType.DMA] * (3 * UN),
  )
  def k(*refs):
    xls = refs[:H]
    xrs = refs[H:2 * H]
    ems = refs[2 * H:3 * H]
    att_h, src_h, dst_h = refs[3 * H:3 * H + 3]
    sparts = refs[3 * H + 3:3 * H + 3 + H]
    rest = refs[3 * H + 3 + H:]
    attb, idxs, idxd, emb = rest[0], rest[1], rest[2], rest[3]
    xlbs = rest[4:4 + UN]
    xrbs = rest[4 + UN:4 + 2 * UN]
    sbufs = rest[4 + 2 * UN:4 + 3 * UN]
    lsems = rest[4 + 3 * UN:4 + 4 * UN]
    rsems = rest[4 + 4 * UN:4 + 5 * UN]
    wsems = rest[4 + 5 * UN:4 + 6 * UN]
    cid = lax.axis_index("c")
    sid = lax.axis_index("s")
    wid = sid * NC + cid

    pltpu.sync_copy(att_h, attb)
    pltpu.sync_copy(src_h.at[wid], idxs)
    pltpu.sync_copy(dst_h.at[wid], idxd)
    for h in range(H):
      attk = [attb[h, pl.ds(kk * 16, 16)] for kk in range(nkc)]

      def jbody(jg, _):
        row0 = wid * (E // (NC * NS)) + jg * (UN * CE)
        dl = [pltpu.async_copy(xls[h].at[idxs.at[jg * UN + u]], xlbs[u],
                               lsems[u]) for u in range(UN)]
        dr = [pltpu.async_copy(xrs[h].at[idxd.at[jg * UN + u]], xrbs[u],
                               rsems[u]) for u in range(UN)]
        pltpu.sync_copy(ems[h].at[pl.ds(row0, UN * CE)], emb)
        ws = []
        for u in range(UN):
          dl[u].wait()
          dr[u].wait()

          def rbody(r, _, u=u):
            acc = jnp.zeros((16,), f32)
            for kk in range(nkc):
              z = (xlbs[u][r, pl.ds(kk * 16, 16)]
                   + xrbs[u][r, pl.ds(kk * 16, 16)]
                   + emb[u * CE + r, pl.ds(kk * 16, 16)])
              z = jnp.maximum(z, 0.2 * z)
              acc = acc + z * attk[kk]
            sbufs[u][r] = acc
            return 0

          lax.fori_loop(0, CE, rbody, 0)
          ws.append(pltpu.async_copy(
              sbufs[u], sparts[h].at[pl.ds(row0 + u * CE, CE)], wsems[u]))
        for w in ws:
          w.wait()
        return 0

      lax.fori_loop(0, NJ32 // UN, jbody, 0)

  return k


def _make_segsum(H):
  @functools.partial(
      pl.kernel,
      out_type=[jax.ShapeDtypeStruct((NP, 16), f32) for _ in range(H)],
      mesh=_mesh(),
      compiler_params=_SC_PARAMS,
      scratch_types=[
          pltpu.VMEM((NJ16, CE16), i32),
          pltpu.VMEM((UN * CE16, 16), f32),
          pltpu.VMEM_SHARED((NP, 16), f32),
      ] + [pltpu.SemaphoreType.DMA] * UN,
  )
  def k(*refs):
    e_hs = refs[:H]
    dst_h, z5_h = refs[H:H + 2]
    dens = refs[H + 2:H + 2 + H]
    idxd, ebuf, tbl = refs[H + 2 + H:H + 5 + H]
    ssems = refs[H + 5 + H:]
    cid = lax.axis_index("c")
    sid = lax.axis_index("s")
    pltpu.sync_copy(dst_h.at[sid], idxd)

    def job(h):
      base = sid * RSNP
      pltpu.sync_copy(z5_h.at[pl.ds(0, RSNP)], tbl.at[pl.ds(base, RSNP)])
      plsc.subcore_barrier()

      def jbody(jg, _):
        row0 = sid * (E // NS) + jg * (UN * CE16)
        pltpu.sync_copy(e_hs[h].at[pl.ds(row0, UN * CE16)], ebuf)
        ss = [pltpu.async_copy(ebuf.at[pl.ds(u * CE16, CE16)],
                               tbl.at[idxd.at[jg * UN + u]],
                               ssems[u], add=True) for u in range(UN)]
        for s in ss:
          s.wait()
        return 0

      lax.fori_loop(0, NJ16 // UN, jbody, 0)
      plsc.subcore_barrier()
      pltpu.sync_copy(tbl.at[pl.ds(base, RSNP)],
                      dens[h].at[pl.ds(base, RSNP)])
      plsc.subcore_barrier()

    @pl.when(cid == 0)
    def _():
      for h in range(H // 2):
        job(h)

    @pl.when(cid == 1)
    def _():
      for h in range(H // 2, H):
        job(h)

  return k


def _make_wsum(H, Cp, HV):
  Cw = Cp // HV
  nkc = Cw // 16
  NJOB = H * HV

  @functools.partial(
      pl.kernel,
      out_type=[jax.ShapeDtypeStruct((NP, Cw), f32) for _ in range(NJOB)],
      mesh=_mesh(),
      compiler_params=_SC_PARAMS,
      scratch_types=[
          pltpu.VMEM((NJ16, CE16), i32),
          pltpu.VMEM((NJ16, CE16), i32),
          pltpu.VMEM((UN * CE16, 16), f32),
      ] + [pltpu.VMEM((CE16, 16), f32) for _ in range(UN)] +
      [pltpu.VMEM((CE16, Cw), f32) for _ in range(UN)] + [
          pltpu.VMEM_SHARED((NP, Cw), f32),
      ] + [pltpu.SemaphoreType.DMA] * (3 * UN),
  )
  def k(*refs):
    xls = refs[:NJOB]
    dens = refs[NJOB:NJOB + H]
    e_hs = refs[NJOB + H:NJOB + 2 * H]
    src_h, dst_h, zc_h = refs[NJOB + 2 * H:NJOB + 2 * H + 3]
    outs = refs[NJOB + 2 * H + 3:2 * NJOB + 2 * H + 3]
    rest = refs[2 * NJOB + 2 * H + 3:]
    idxs, idxd, ebuf = rest[0], rest[1], rest[2]
    dbufs = rest[3:3 + UN]
    xbufs = rest[3 + UN:3 + 2 * UN]
    tbl = rest[3 + 2 * UN]
    dsems = rest[4 + 2 * UN:4 + 3 * UN]
    xsems = rest[4 + 3 * UN:4 + 4 * UN]
    ssems = rest[4 + 4 * UN:4 + 5 * UN]
    cid = lax.axis_index("c")
    sid = lax.axis_index("s")
    pltpu.sync_copy(src_h.at[sid], idxs)
    pltpu.sync_copy(dst_h.at[sid], idxd)

    def job(jj):
      h = jj // HV
      base = sid * RSNP
      pltpu.sync_copy(zc_h, tbl.at[pl.ds(base, RSNP)])
      plsc.subcore_barrier()

      def jbody(jg, _):
        row0 = sid * (E // NS) + jg * (UN * CE16)
        dd = [pltpu.async_copy(dens[h].at[idxd.at[jg * UN + u]], dbufs[u],
                               dsems[u]) for u in range(UN)]
        dx = [pltpu.async_copy(xls[jj].at[idxs.at[jg * UN + u]], xbufs[u],
                               xsems[u]) for u in range(UN)]
        pltpu.sync_copy(e_hs[h].at[pl.ds(row0, UN * CE16)], ebuf)
        ss = []
        for u in range(UN):
          dd[u].wait()
          dx[u].wait()

          def rbody(r, _, u=u):
            av = ebuf[u * CE16 + r] / jnp.maximum(dbufs[u][r], 1e-30)
            for kk in range(nkc):
              xbufs[u][r, pl.ds(kk * 16, 16)] = (
                  xbufs[u][r, pl.ds(kk * 16, 16)] * av)
            return 0

          lax.fori_loop(0, CE16, rbody, 0)
          ss.append(pltpu.async_copy(xbufs[u], tbl.at[idxd.at[jg * UN + u]],
                                     ssems[u], add=True))
        for s in ss:
          s.wait()
        return 0

      lax.fori_loop(0, NJ16 // UN, jbody, 0)
      plsc.subcore_barrier()
      pltpu.sync_copy(tbl.at[pl.ds(base, RSNP)],
                      outs[jj].at[pl.ds(base, RSNP)])
      plsc.subcore_barrier()

    @pl.when(cid == 0)
    def _():
      for jj in range(NJOB // 2):
        job(jj)

    @pl.when(cid == 1)
    def _():
      for jj in range(NJOB // 2, NJOB):
        job(jj)

  return k


def _dot(a, b):
  return lax.dot_general(a, b, (((1,), (0,)), ((), ())),
                         preferred_element_type=f32)


def _mm(A, W, bias=None, relu=False, bm=400):
  M, K = A.shape
  Nc = W.shape[1]
  grid = (M // bm,)
  in_specs = [
      pl.BlockSpec((bm, K), lambda i: (i, 0)),
      pl.BlockSpec((K, Nc), lambda i: (0, 0)),
  ]
  if bias is not None:
    in_specs.append(pl.BlockSpec((8, Nc), lambda i: (0, 0)))

  def body(*refs):
    a_ref, w_ref = refs[0], refs[1]
    o_ref = refs[-1]
    acc = _dot(a_ref[...], w_ref[...])
    if bias is not None:
      acc = acc + refs[2][0:1, :]
    if relu:
      acc = jnp.maximum(acc, 0.0)
    o_ref[...] = acc

  fn = pl.pallas_call(
      body, grid=grid, in_specs=in_specs,
      out_specs=pl.BlockSpec((bm, Nc), lambda i: (i, 0)),
      out_shape=jax.ShapeDtypeStruct((M, Nc), f32))
  args = (A, W) if bias is None else (A, W, bias)
  return fn(*args)


def _mm_multi(A, Ws, bm=400):
  M, K = A.shape
  Cp = Ws[0].shape[1]
  nw = len(Ws)

  def body(*refs):
    a = refs[0]
    ws = refs[1:1 + nw]
    outs = refs[1 + nw:]
    av = a[...]
    for h in range(nw):
      outs[h][...] = _dot(av, ws[h][...])

  return pl.pallas_call(
      body, grid=(M // bm,),
      in_specs=[pl.BlockSpec((bm, K), lambda i: (i, 0))] +
               [pl.BlockSpec((K, Cp), lambda i: (0, 0))] * nw,
      out_specs=[pl.BlockSpec((bm, Cp), lambda i: (i, 0))] * nw,
      out_shape=[jax.ShapeDtypeStruct((M, Cp), f32)] * nw)(A, *Ws)


def _rgcn_mm(h_p, Wroot_p, aggf, cntfa, cntfb, Wrel_f, brep, bm=400):
  M, K1 = h_p.shape
  K2 = aggf.shape[1]
  Nc = Wroot_p.shape[1]

  def body(a1, w1, a2, ca, cb, w2, b, o):
    mean = a2[...] / jnp.maximum(ca[...] + cb[...], 1.0)
    acc = _dot(a1[...], w1[...]) + _dot(mean, w2[...]) + b[0:1, :]
    o[...] = jnp.maximum(acc, 0.0)

  return pl.pallas_call(
      body, grid=(M // bm,),
      in_specs=[
          pl.BlockSpec((bm, K1), lambda i: (i, 0)),
          pl.BlockSpec((K1, Nc), lambda i: (0, 0)),
          pl.BlockSpec((bm, K2), lambda i: (i, 0)),
          pl.BlockSpec((bm, K2), lambda i: (i, 0)),
          pl.BlockSpec((bm, K2), lambda i: (i, 0)),
          pl.BlockSpec((K2, Nc), lambda i: (0, 0)),
          pl.BlockSpec((8, Nc), lambda i: (0, 0)),
      ],
      out_specs=pl.BlockSpec((bm, Nc), lambda i: (i, 0)),
      out_shape=jax.ShapeDtypeStruct((M, Nc), f32))(
          h_p, Wroot_p, aggf, cntfa, cntfb, Wrel_f, brep)


def _bias_relu(A, brep, bm=400):
  M, W = A.shape

  def body(a, b, o):
    o[...] = jnp.maximum(a[...] + b[0:1, :], 0.0)

  return pl.pallas_call(
      body, grid=(M // bm,),
      in_specs=[pl.BlockSpec((bm, W), lambda i: (i, 0)),
                pl.BlockSpec((8, W), lambda i: (0, 0))],
      out_specs=pl.BlockSpec((bm, W), lambda i: (i, 0)),
      out_shape=jax.ShapeDtypeStruct((M, W), f32))(A, brep)


def _colmax(sparts, ones16, bm=1600):
  H = len(sparts)

  def body(*refs):
    i = pl.program_id(0)
    ss = refs[:H]
    b = refs[H]
    o = refs[H + 1]

    @pl.when(i == 0)
    def _():
      o[...] = jnp.full((8, H * 16), -3e38, f32)

    for h in range(H):
      srep = _dot(ss[h][...], b[...])
      m = jnp.max(srep, axis=0, keepdims=True)
      o[:, h * 16:(h + 1) * 16] = jnp.maximum(
          o[:, h * 16:(h + 1) * 16], jnp.broadcast_to(m, (8, 16)))

  return pl.pallas_call(
      body, grid=(E // bm,),
      in_specs=[pl.BlockSpec((bm, 16), lambda i: (i, 0))] * H +
               [pl.BlockSpec((16, 16), lambda i: (0, 0))],
      out_specs=pl.BlockSpec((8, H * 16), lambda i: (0, 0)),
      out_shape=jax.ShapeDtypeStruct((8, H * 16), f32))(*sparts, ones16)


def _exp_e(sparts, ones16, mrep, bm=1600):
  H = len(sparts)

  def body(*refs):
    ss = refs[:H]
    b = refs[H]
    m = refs[H + 1]
    outs = refs[H + 2:]
    for h in range(H):
      srep = _dot(ss[h][...], b[...])
      outs[h][...] = jnp.exp(srep - m[0:1, h * 16:(h + 1) * 16])

  return pl.pallas_call(
      body, grid=(E // bm,),
      in_specs=[pl.BlockSpec((bm, 16), lambda i: (i, 0))] * H +
               [pl.BlockSpec((16, 16), lambda i: (0, 0)),
                pl.BlockSpec((8, H * 16), lambda i: (0, 0))],
      out_specs=[pl.BlockSpec((bm, 16), lambda i: (i, 0))] * H,
      out_shape=[jax.ShapeDtypeStruct((E, 16), f32)] * H)(
          *sparts, ones16, mrep)


WIN = 512


def _pool(x, batch2d, starts):
  W = x.shape[1]

  def body(st_ref, x_ref, b_ref, o_ref):
    def gbody(g, _):
      s0 = pl.multiple_of((jnp.minimum(st_ref[g], N - WIN) // 8) * 8, 8)
      xw = x_ref[pl.ds(s0, WIN), :]
      bw = b_ref[pl.ds(s0, WIN), 0:1]
      mask = bw == g
      mx = jnp.max(jnp.where(mask, xw, f32(-3e38)), axis=0, keepdims=True)
      mx = jnp.where(mx <= f32(-1e38), 0.0, mx)
      sm = jnp.sum(jnp.where(mask, xw, 0.0), axis=0, keepdims=True)
      cnt = (st_ref[g + 1] - st_ref[g]).astype(f32)
      mean = sm / jnp.maximum(cnt, 1.0)
      o_ref[pl.ds(g, 1), :] = jnp.concatenate([mx, mean], axis=1)
      return 0

    lax.fori_loop(0, G, gbody, 0)

  return pl.pallas_call(
      body,
      in_specs=[pl.BlockSpec(memory_space=pltpu.SMEM),
                pl.BlockSpec((N, W), lambda: (0, 0)),
                pl.BlockSpec((N, 128), lambda: (0, 0))],
      out_specs=pl.BlockSpec((G, 2 * W), lambda: (0, 0)),
      out_shape=jax.ShapeDtypeStruct((G, 2 * W), f32))(starts, x, batch2d)


def _fc_ln(h, Wfc, brep, grep, berep):
  def body(h_ref, w_ref, b_ref, g_ref, be_ref, o_ref):
    z = _dot(h_ref[...], w_ref[...]) + b_ref[0:1, :]
    z = jnp.maximum(z, 0.0)
    mu = jnp.mean(z, axis=-1, keepdims=True)
    d = z - mu
    var = jnp.mean(d * d, axis=-1, keepdims=True)
    o_ref[...] = d / jnp.sqrt(var + 1e-5) * g_ref[0:1, :] + be_ref[0:1, :]

  return pl.pallas_call(
      body, out_shape=jax.ShapeDtypeStruct((G, 1024), f32))(
          h, Wfc, brep, grep, berep)


def _pad2(a, r, c):
  return jnp.pad(a, ((0, r - a.shape[0]), (0, c - a.shape[1])))


def _rep8(v, w):
  return jnp.broadcast_to(jnp.pad(v, (0, w - v.shape[0]))[None, :], (8, w))


def _headpad_cols(Wm, H, C, Cp, rpad):
  K = Wm.shape[0]
  w = jnp.pad(Wm.reshape(K, H, C), ((0, rpad - K), (0, 0), (0, Cp - C)))
  return w.reshape(rpad, H * Cp)


def _headpad_rows(Wm, Hi, Ci, Cip):
  Nc = Wm.shape[1]
  w = jnp.pad(Wm.reshape(Hi, Ci, Nc), ((0, 0), (0, Cip - Ci), (0, 0)))
  return w.reshape(Hi * Cip, Nc)


def kernel(x, edge_attr, W_ee, b_ee, Wrel1, Wroot1, b1, Wrel2, Wroot2, b2,
           Wrel3, Wroot3, b3, Wl1, Wr1, We1, att1, bg1, Wl2, Wr2, We2, att2,
           bg2, Wfc, bfc, gamma, beta, edge_index, edge_type, batch):
  src = edge_index[0].astype(i32)
  dst = edge_index[1].astype(i32)
  dst5 = dst * NREL + edge_type.astype(i32)

  src32 = src.reshape(NC * NS, NJ32, CE)
  dst32 = dst.reshape(NC * NS, NJ32, CE)
  src16 = src.reshape(NS, NJ16, CE16)
  dst16 = dst.reshape(NS, NJ16, CE16)
  dst5_16 = dst5.reshape(NS, NJ16, CE16)

  z5 = jnp.zeros((RS5P, 16), f32)
  ones_in = jnp.ones((CE16, 16), f32)
  ones16 = jnp.ones((16, 16), f32)

  cntA, cntB = _make_count()(dst5_16, ones_in, z5)
  cnt5a = cntA[:N5, 0].reshape(N, NREL)
  cnt5b = cntB[:N5, 0].reshape(N, NREL)

  x_p = _pad2(x, N, 80)
  agg5 = _make_agg(5)
  agg10 = _make_agg(10)

  def rgcn(h_p, Pin, agg, Wroot, b, Wrel, Pout):
    hs = [h_p[:, 16 * kk:16 * kk + 16] for kk in range(Pin // 16)]
    sums = agg(*hs, src16, dst5_16, z5)
    aggf = jnp.concatenate(
        [s[:N5].reshape(N, NREL, 16) for s in sums], axis=2
    ).reshape(N, NREL * Pin)
    cntfa = jnp.broadcast_to(cnt5a[:, :, None], (N, NREL, Pin)).reshape(
        N, NREL * Pin)
    cntfb = jnp.broadcast_to(cnt5b[:, :, None], (N, NREL, Pin)).reshape(
        N, NREL * Pin)
    Din, Dout = Wroot.shape
    Wroot_p = _pad2(Wroot, Pin, Pout)
    Wrel_p = jnp.pad(Wrel, ((0, 0), (0, Pin - Din), (0, Pout - Dout)))
    Wrel_f = Wrel_p.reshape(NREL * Pin, Pout)
    return _rgcn_mm(h_p, Wroot_p, aggf, cntfa, cntfb, Wrel_f, _rep8(b, Pout))

  h1 = rgcn(x_p, 80, agg5, Wroot1, b1, Wrel1, 80)
  h2 = rgcn(h1, 80, agg5, Wroot2, b2, Wrel2, 160)
  x3 = rgcn(h2, 160, agg10, Wroot3, b3, Wrel3, 320)

  batch_i = batch.astype(i32)
  starts = jnp.searchsorted(batch_i, jnp.arange(G + 1, dtype=i32)).astype(i32)
  batch2d = jnp.broadcast_to(batch_i[:, None], (N, 128))
  p1 = _pool(x3, batch2d, starts)

  ee = _mm(edge_attr, _pad2(W_ee, 8, 128), bias=_rep8(b_ee, 128), relu=True,
           bm=1600)

  def gat(x_in, Pi, H, C, Cp, Wl, Wr, We, att, bg, Hi, Ci):
    Wl_r = _headpad_rows(Wl, Hi, Ci, Pi // Hi)
    Wr_r = _headpad_rows(Wr, Hi, Ci, Pi // Hi)
    Wl_hp = _headpad_cols(Wl_r, H, C, Cp, Pi)
    Wr_hp = _headpad_cols(Wr_r, H, C, Cp, Pi)
    We_hp = _headpad_cols(We, H, C, Cp, 128)
    Wlr_heads = ([Wl_hp[:, h * Cp:(h + 1) * Cp] for h in range(H)] +
                 [Wr_hp[:, h * Cp:(h + 1) * Cp] for h in range(H)])
    xlr = _mm_multi(x_in, Wlr_heads)
    xls, xrs = xlr[:H], xlr[H:]
    ems = _mm_multi(ee, [We_hp[:, h * Cp:(h + 1) * Cp] for h in range(H)],
                    bm=1600)
    att_p = jnp.pad(att, ((0, 8 - H), (0, Cp - C)))
    sparts = _make_score(H, Cp)(*xls, *xrs, *ems, att_p, src32, dst32)
    mrep = _colmax(sparts, ones16)
    e_hs = _exp_e(sparts, ones16, mrep)
    dens = _make_segsum(H)(*e_hs, dst16, z5)
    HV = 2 if Cp > 80 else 1
    Cw = Cp // HV
    zc = jnp.zeros((RSNP, Cw), f32)
    xls_h = [xl[:, v * Cw:(v + 1) * Cw] for xl in xls for v in range(HV)]
    outg = _make_wsum(H, Cp, HV)(*xls_h, *dens, *e_hs, src16, dst16, zc)
    outg_cat = jnp.concatenate([o[:N] for o in outg], axis=1)
    bg_hp = jnp.pad(bg.reshape(H, C), ((0, 0), (0, Cp - C))).reshape(H * Cp)
    return _bias_relu(outg_cat, jnp.broadcast_to(bg_hp[None, :], (8, H * Cp)))

  x2 = gat(x_p, 80, 2, D, 80, Wl1, Wr1, We1, att1, bg1, 1, 78)
  x2b = gat(x2, 160, 4, 2 * D, 160, Wl2, Wr2, We2, att2, bg2, 2, 78)
  p2 = _pool(x2b, batch2d, starts)

  p1mx = p1[:, 0:312]
  p1me = p1[:, 320:632]
  p2mx = jnp.concatenate([p2[:, h * 160:h * 160 + 156] for h in range(4)], 1)
  p2me = jnp.concatenate(
      [p2[:, 640 + h * 160:640 + h * 160 + 156] for h in range(4)], 1)
  hcat = jnp.concatenate([p1mx, p1me, p2mx, p2me], axis=1)
  return _fc_ln(hcat, Wfc, _rep8(bfc, 1024), _rep8(gamma, 1024),
                _rep8(beta, 1024))
